# Initial kernel scaffold; baseline (speedup 1.0000x reference)
import jax
import jax.numpy as jnp
from jax.experimental import pallas as pl


def kernel(indexes, table, W):
    raise NotImplementedError("write your pallas kernel here")


# speedup vs baseline: 13.3088x; 13.3088x over previous
import functools

import jax
import jax.numpy as jnp
from jax import lax
from jax.experimental import pallas as pl
from jax.experimental.pallas import tpu as pltpu
from jax.experimental.pallas import tpu_sc as plsc

DIM = 32
NC, NS = 2, 16
NW = NC * NS
IDX_CHUNK = 128
GATHERS_PER_STEP = 13
STEP_ROWS = IDX_CHUNK * GATHERS_PER_STEP


def _sc_gather(table, idx2d, n_rows):
    per_w = n_rows // NW
    idx_rows_per_w = per_w // IDX_CHUNK
    steps = per_w // STEP_ROWS
    mesh = plsc.VectorSubcoreMesh(core_axis_name="c", subcore_axis_name="s")

    @functools.partial(
        pl.kernel,
        mesh=mesh,
        compiler_params=pltpu.CompilerParams(use_tc_tiling_on_sc=False),
        out_type=jax.ShapeDtypeStruct((n_rows, DIM), jnp.float32),
        scratch_types=[
            pltpu.VMEM((idx_rows_per_w, IDX_CHUNK), jnp.int32),
            pltpu.VMEM((STEP_ROWS, DIM), jnp.float32),
            pltpu.SemaphoreType.DMA,
        ],
    )
    def k(table_hbm, idx_hbm, out_hbm, idx_v, rows_v, sem):
        wid = lax.axis_index("s") * NC + lax.axis_index("c")
        pltpu.sync_copy(idx_hbm.at[pl.ds(wid * idx_rows_per_w, idx_rows_per_w)],
                        idx_v)
        row_base = wid * per_w

        def step(s, carry):
            copies = []
            for j in range(GATHERS_PER_STEP):
                copies.append(pltpu.async_copy(
                    table_hbm.at[idx_v.at[s * GATHERS_PER_STEP + j]],
                    rows_v.at[pl.ds(j * IDX_CHUNK, IDX_CHUNK)],
                    sem,
                ))
            for c in copies:
                c.wait()
            pltpu.sync_copy(
                rows_v,
                out_hbm.at[pl.ds(row_base + s * STEP_ROWS, STEP_ROWS)])
            return carry

        lax.fori_loop(0, steps, step, 0)

    return k(table, idx2d)


def _tc_project(emb4, wbig):
    m = emb4.shape[0]
    bm = 4096
    grid = m // bm

    def body(x_ref, w_ref, o_ref):
        o_ref[...] = jnp.dot(x_ref[...], w_ref[...],
                             preferred_element_type=jnp.float32)

    return pl.pallas_call(
        body,
        grid=(grid,),
        in_specs=[pl.BlockSpec((bm, 128), lambda i: (i, 0)),
                  pl.BlockSpec((128, 128), lambda i: (0, 0))],
        out_specs=pl.BlockSpec((bm, 128), lambda i: (i, 0)),
        out_shape=jax.ShapeDtypeStruct((m, 128), jnp.float32),
    )(emb4, wbig)


def kernel(indexes, table, W):
    b, f = indexes.shape
    n = b * f
    idx2d = indexes.reshape(n // IDX_CHUNK, IDX_CHUNK).astype(jnp.int32)
    emb = _sc_gather(table, idx2d, n)
    wbig = jnp.kron(jnp.eye(4, dtype=jnp.float32), W.T)
    out4 = _tc_project(emb.reshape(n // 4, 128), wbig)
    return out4.reshape(b, f, DIM)
